# Initial kernel scaffold; baseline (speedup 1.0000x reference)
import jax
import jax.numpy as jnp
from jax.experimental import pallas as pl


def kernel(x, edge_index, W1, b1, W2, b2):
    raise NotImplementedError("write your pallas kernel here")


# speedup vs baseline: 12.0568x; 12.0568x over previous
import functools

import jax
import jax.numpy as jnp
from jax import lax
from jax.experimental import pallas as pl
from jax.experimental.pallas import tpu as pltpu
from jax.experimental.pallas import tpu_sc as plsc

NC = 2
NS = 16
NW = NC * NS
CH = 128
LANES = 16


def _cdiv(a, b):
    return (a + b - 1) // b


def _sc_degree(dst2d, n_pad, k_per_tile):
    mesh = plsc.VectorSubcoreMesh(core_axis_name="c", subcore_axis_name="s")
    zslice = n_pad // NS

    @functools.partial(
        pl.kernel,
        out_type=jax.ShapeDtypeStruct((NC * n_pad,), jnp.float32),
        mesh=mesh,
        scratch_types=[
            pltpu.VMEM((CH,), jnp.int32),
            pltpu.VMEM((CH,), jnp.float32),
            pltpu.VMEM((zslice,), jnp.float32),
            pltpu.VMEM_SHARED((n_pad,), jnp.float32),
        ],
    )
    def deg_kernel(dst_hbm, out_hbm, didx, ones, zbuf, acc):
        c = lax.axis_index("c")
        s = lax.axis_index("s")
        wid = c * NS + s

        def fill_ones(i, carry):
            ones[pl.ds(i * LANES, LANES)] = jnp.full((LANES,), 1.0, jnp.float32)
            return carry

        lax.fori_loop(0, CH // LANES, fill_ones, 0)

        def fill_zero(i, carry):
            zbuf[pl.ds(i * LANES, LANES)] = jnp.zeros((LANES,), jnp.float32)
            return carry

        lax.fori_loop(0, zslice // LANES, fill_zero, 0)
        pltpu.sync_copy(zbuf, acc.at[pl.ds(s * zslice, zslice)])
        plsc.subcore_barrier()

        def body(k, carry):
            cid = wid * k_per_tile + k
            pltpu.sync_copy(dst_hbm.at[cid], didx)
            pltpu.sync_copy(ones, acc.at[didx], add=True)
            return carry

        lax.fori_loop(0, k_per_tile, body, 0)
        plsc.subcore_barrier()
        pltpu.sync_copy(acc.at[pl.ds(s * zslice, zslice)],
                        out_hbm.at[pl.ds(c * n_pad + s * zslice, zslice)])

    return deg_kernel(dst2d)


def _sc_aggregate(src2d, dst2d, y, n, n_pad, k_per_tile):
    d = y.shape[1]
    mesh = plsc.VectorSubcoreMesh(core_axis_name="c", subcore_axis_name="s")
    zslice = n_pad // NS

    @functools.partial(
        pl.kernel,
        out_type=jax.ShapeDtypeStruct((NC * n_pad, d), jnp.float32),
        mesh=mesh,
        scratch_types=[
            pltpu.VMEM((CH,), jnp.int32),
            pltpu.VMEM((CH,), jnp.int32),
            pltpu.VMEM((CH, d), jnp.float32),
            pltpu.VMEM_SHARED((n_pad, d), jnp.float32),
            pltpu.SemaphoreType.DMA,
        ],
        compiler_params=pltpu.CompilerParams(use_tc_tiling_on_sc=False),
    )
    def agg_kernel(src_hbm, dst_hbm, y_hbm, out_hbm, sidx, didx, rows, acc, sem):
        c = lax.axis_index("c")
        s = lax.axis_index("s")
        wid = c * NS + s

        def fill_zero(i, carry):
            j = i // (d // LANES)
            l = i % (d // LANES)
            rows[j, pl.ds(l * LANES, LANES)] = jnp.zeros((LANES,), jnp.float32)
            return carry

        lax.fori_loop(0, CH * d // LANES, fill_zero, 0)

        def zero_acc(j, carry):
            pltpu.sync_copy(rows, acc.at[pl.ds(s * zslice + j * CH, CH)])
            return carry

        lax.fori_loop(0, zslice // CH, zero_acc, 0)
        plsc.subcore_barrier()

        def body(k, carry):
            cid = wid * k_per_tile + k
            pltpu.sync_copy(src_hbm.at[cid], sidx)
            pltpu.sync_copy(dst_hbm.at[cid], didx)
            pltpu.async_copy(y_hbm.at[sidx], rows, sem).wait()
            pltpu.sync_copy(rows, acc.at[didx], add=True)
            return carry

        lax.fori_loop(0, k_per_tile, body, 0)
        plsc.subcore_barrier()
        pltpu.sync_copy(acc.at[pl.ds(s * zslice, zslice)],
                        out_hbm.at[pl.ds(c * n_pad + s * zslice, zslice)])

    return agg_kernel(src2d, dst2d, y)


def _tc_dinv(parts3):
    def body(p_ref, o_ref):
        deg = p_ref[0] + p_ref[1] + 1.0
        o_ref[...] = lax.rsqrt(deg)

    return pl.pallas_call(
        body,
        out_shape=jax.ShapeDtypeStruct(parts3.shape[1:], jnp.float32),
    )(parts3)


def _tc_matmul_scale(x, w, dinv_col):
    def body(x_ref, w_ref, v_ref, o_ref):
        o_ref[...] = jnp.dot(x_ref[...], w_ref[...],
                             preferred_element_type=jnp.float32) * v_ref[...]

    return pl.pallas_call(
        body,
        out_shape=jax.ShapeDtypeStruct((x.shape[0], w.shape[1]), jnp.float32),
    )(x, w, dinv_col)


def _tc_mid(acc_a, acc_b, y1, dinv_col, b1row, w2):
    def body(a_ref, b_ref, y_ref, v_ref, bias_ref, w_ref, o_ref):
        h = (a_ref[...] + b_ref[...] + y_ref[...]) * v_ref[...] + bias_ref[...]
        h = jnp.maximum(h, 0.0)
        o_ref[...] = jnp.dot(h, w_ref[...],
                             preferred_element_type=jnp.float32) * v_ref[...]

    return pl.pallas_call(
        body,
        out_shape=jax.ShapeDtypeStruct((y1.shape[0], w2.shape[1]), jnp.float32),
    )(acc_a, acc_b, y1, dinv_col, b1row, w2)


def _tc_final(acc_a, acc_b, y2, dinv_col, b2row):
    def body(a_ref, b_ref, y_ref, v_ref, bias_ref, o_ref):
        o_ref[...] = ((a_ref[...] + b_ref[...] + y_ref[...]) * v_ref[...]
                      + bias_ref[...])

    return pl.pallas_call(
        body,
        out_shape=jax.ShapeDtypeStruct(y2.shape, jnp.float32),
    )(acc_a, acc_b, y2, dinv_col, b2row)


def kernel(x, edge_index, W1, b1, W2, b2):
    n = x.shape[0]
    e = edge_index.shape[1]

    n_chunks = _cdiv(e, CH)
    k_per_tile = _cdiv(n_chunks, NW)
    e_pad = k_per_tile * NW * CH
    src = edge_index[0].astype(jnp.int32)
    dst = edge_index[1].astype(jnp.int32)
    pad = e_pad - e
    src2d = jnp.concatenate(
        [src, jnp.zeros((pad,), jnp.int32)]).reshape(k_per_tile * NW, CH)
    dst2d = jnp.concatenate(
        [dst, jnp.full((pad,), n, jnp.int32)]).reshape(k_per_tile * NW, CH)

    n_pad = _cdiv(n + 1, NS * CH) * NS * CH

    deg_parts = _sc_degree(dst2d, n_pad, k_per_tile)
    dinv2d = _tc_dinv(deg_parts.reshape(NC, n_pad // 128, 128))
    dinv_col = dinv2d.reshape(n_pad)[:n].reshape(n, 1)

    y1 = _tc_matmul_scale(x, W1, dinv_col)
    agg1 = _sc_aggregate(src2d, dst2d, y1, n, n_pad, k_per_tile)
    y2 = _tc_mid(agg1[:n], agg1[n_pad:n_pad + n], y1, dinv_col,
                 b1.reshape(1, -1), W2)
    agg2 = _sc_aggregate(src2d, dst2d, y2, n, n_pad, k_per_tile)
    out = _tc_final(agg2[:n], agg2[n_pad:n_pad + n], y2, dinv_col,
                    b2.reshape(1, -1))
    return out


# speedup vs baseline: 21.8950x; 1.8160x over previous
import functools

import jax
import jax.numpy as jnp
from jax import lax
from jax.experimental import pallas as pl
from jax.experimental.pallas import tpu as pltpu
from jax.experimental.pallas import tpu_sc as plsc

NC = 2
NS = 16
NW = NC * NS
CH = 128
LANES = 16


def _cdiv(a, b):
    return (a + b - 1) // b


def _sc_degree(dst2d, n_pad, k_per_tile):
    mesh = plsc.VectorSubcoreMesh(core_axis_name="c", subcore_axis_name="s")
    zslice = n_pad // NS

    @functools.partial(
        pl.kernel,
        out_type=jax.ShapeDtypeStruct((NC * n_pad,), jnp.float32),
        mesh=mesh,
        scratch_types=[
            pltpu.VMEM((k_per_tile, CH), jnp.int32),
            pltpu.VMEM((CH,), jnp.float32),
            pltpu.VMEM((_cdiv(zslice, LANES) * LANES,), jnp.float32),
            pltpu.VMEM_SHARED((n_pad,), jnp.float32),
            pltpu.SemaphoreType.DMA,
        ],
        compiler_params=pltpu.CompilerParams(use_tc_tiling_on_sc=False),
    )
    def deg_kernel(dst_hbm, out_hbm, didx, ones, zbuf, acc, sem):
        c = lax.axis_index("c")
        s = lax.axis_index("s")
        wid = c * NS + s

        def fill_ones(i, carry):
            ones[pl.ds(i * LANES, LANES)] = jnp.full((LANES,), 1.0, jnp.float32)
            return carry

        lax.fori_loop(0, CH // LANES, fill_ones, 0)

        def fill_zero(i, carry):
            zbuf[pl.ds(i * LANES, LANES)] = jnp.zeros((LANES,), jnp.float32)
            return carry

        lax.fori_loop(0, _cdiv(zslice, LANES), fill_zero, 0)
        pltpu.sync_copy(zbuf.at[pl.ds(0, zslice)],
                        acc.at[pl.ds(s * zslice, zslice)])
        pltpu.sync_copy(dst_hbm.at[pl.ds(wid * k_per_tile, k_per_tile)], didx)
        plsc.subcore_barrier()

        BATCH = 8

        def batch(q, carry):
            for b in range(BATCH):
                k = q * BATCH + b

                @pl.when(k < k_per_tile)
                def _():
                    pltpu.async_copy(ones, acc.at[didx.at[k]], sem, add=True)
            for b in range(BATCH):
                k = q * BATCH + b

                @pl.when(k < k_per_tile)
                def _():
                    pltpu.make_async_copy(ones, acc.at[didx.at[k]],
                                          sem).wait()
            return carry

        lax.fori_loop(0, _cdiv(k_per_tile, BATCH), batch, 0)
        plsc.subcore_barrier()
        pltpu.sync_copy(acc.at[pl.ds(s * zslice, zslice)],
                        out_hbm.at[pl.ds(c * n_pad + s * zslice, zslice)])

    return deg_kernel(dst2d)


def _sc_aggregate(src2d, dst2d, y, n, n_pad, k_per_tile, ch, nbuf):
    d = y.shape[1]
    mesh = plsc.VectorSubcoreMesh(core_axis_name="c", subcore_axis_name="s")
    zslice = n_pad // NS

    @functools.partial(
        pl.kernel,
        out_type=jax.ShapeDtypeStruct((NC * n_pad, d), jnp.float32),
        mesh=mesh,
        scratch_types=[
            pltpu.VMEM((k_per_tile, ch), jnp.int32),
            pltpu.VMEM((k_per_tile, ch), jnp.int32),
            [pltpu.VMEM((ch, d), jnp.float32)] * nbuf,
            pltpu.VMEM_SHARED((n_pad, d), jnp.float32),
            [pltpu.SemaphoreType.DMA] * nbuf,
            [pltpu.SemaphoreType.DMA] * nbuf,
        ],
        compiler_params=pltpu.CompilerParams(use_tc_tiling_on_sc=False),
    )
    def agg_kernel(src_hbm, dst_hbm, y_hbm, out_hbm, sidx, didx, rows, acc,
                   gsem, ssem):
        c = lax.axis_index("c")
        s = lax.axis_index("s")
        wid = c * NS + s
        base = wid * k_per_tile

        def fill_zero(i, carry):
            j = i // (d // LANES)
            l = i % (d // LANES)
            rows[0][j, pl.ds(l * LANES, LANES)] = jnp.zeros((LANES,),
                                                            jnp.float32)
            return carry

        lax.fori_loop(0, ch * d // LANES, fill_zero, 0)

        def zero_acc(j, carry):
            pltpu.sync_copy(rows[0], acc.at[pl.ds(s * zslice + j * ch, ch)])
            return carry

        lax.fori_loop(0, zslice // ch, zero_acc, 0)
        rem = zslice % ch
        if rem:
            pltpu.sync_copy(
                rows[0].at[pl.ds(0, rem)],
                acc.at[pl.ds(s * zslice + (zslice // ch) * ch, rem)])
        pltpu.sync_copy(src_hbm.at[pl.ds(base, k_per_tile)], sidx)
        pltpu.sync_copy(dst_hbm.at[pl.ds(base, k_per_tile)], didx)
        plsc.subcore_barrier()

        def gather(k, b):
            pltpu.async_copy(y_hbm.at[sidx.at[k]], rows[b], gsem[b])

        def gather_wait(k, b):
            pltpu.make_async_copy(y_hbm.at[sidx.at[k]], rows[b],
                                  gsem[b]).wait()

        def scatter(k, b):
            pltpu.async_copy(rows[b], acc.at[didx.at[k]], ssem[b], add=True)

        def scatter_wait(k, b):
            pltpu.make_async_copy(rows[b], acc.at[didx.at[k]],
                                  ssem[b]).wait()

        for b in range(min(nbuf, k_per_tile)):
            gather(b, b)

        def ring(q, carry):
            for b in range(nbuf):
                k = q * nbuf + b

                @pl.when(k < k_per_tile)
                def _():
                    gather_wait(k, b)
                    scatter(k, b)
            for b in range(nbuf):
                k = q * nbuf + b

                @pl.when(k < k_per_tile)
                def _():
                    scatter_wait(k, b)

                    @pl.when(k + nbuf < k_per_tile)
                    def _():
                        gather(k + nbuf, b)
            return carry

        lax.fori_loop(0, _cdiv(k_per_tile, nbuf), ring, 0)
        plsc.subcore_barrier()
        pltpu.sync_copy(acc.at[pl.ds(s * zslice, zslice)],
                        out_hbm.at[pl.ds(c * n_pad + s * zslice, zslice)])

    return agg_kernel(src2d, dst2d, y)


def _tc_dinv(parts3):
    def body(p_ref, o_ref):
        deg = p_ref[0] + p_ref[1] + 1.0
        o_ref[...] = lax.rsqrt(deg)

    return pl.pallas_call(
        body,
        out_shape=jax.ShapeDtypeStruct(parts3.shape[1:], jnp.float32),
    )(parts3)


def _tc_matmul_scale(x, w, dinv_col):
    def body(x_ref, w_ref, v_ref, o_ref):
        o_ref[...] = jnp.dot(x_ref[...], w_ref[...],
                             preferred_element_type=jnp.float32) * v_ref[...]

    return pl.pallas_call(
        body,
        out_shape=jax.ShapeDtypeStruct((x.shape[0], w.shape[1]), jnp.float32),
    )(x, w, dinv_col)


def _tc_mid(acc_a, acc_b, y1, dinv_col, b1row, w2):
    def body(a_ref, b_ref, y_ref, v_ref, bias_ref, w_ref, o_ref):
        h = (a_ref[...] + b_ref[...] + y_ref[...]) * v_ref[...] + bias_ref[...]
        h = jnp.maximum(h, 0.0)
        o_ref[...] = jnp.dot(h, w_ref[...],
                             preferred_element_type=jnp.float32) * v_ref[...]

    return pl.pallas_call(
        body,
        out_shape=jax.ShapeDtypeStruct((y1.shape[0], w2.shape[1]), jnp.float32),
    )(acc_a, acc_b, y1, dinv_col, b1row, w2)


def _tc_final(acc_a, acc_b, y2, dinv_col, b2row):
    def body(a_ref, b_ref, y_ref, v_ref, bias_ref, o_ref):
        o_ref[...] = ((a_ref[...] + b_ref[...] + y_ref[...]) * v_ref[...]
                      + bias_ref[...])

    return pl.pallas_call(
        body,
        out_shape=jax.ShapeDtypeStruct(y2.shape, jnp.float32),
    )(acc_a, acc_b, y2, dinv_col, b2row)


def kernel(x, edge_index, W1, b1, W2, b2):
    n = x.shape[0]
    e = edge_index.shape[1]

    src = edge_index[0].astype(jnp.int32)
    dst = edge_index[1].astype(jnp.int32)

    def chunked(arr, ch, fill):
        k_per_tile = _cdiv(_cdiv(e, ch), NW)
        padn = k_per_tile * NW * ch - e
        return (jnp.concatenate(
            [arr, jnp.full((padn,), fill, jnp.int32)]).reshape(
                k_per_tile * NW, ch), k_per_tile)

    srcA, kA = chunked(src, 64, 0)
    dstA, _ = chunked(dst, 64, n)
    srcB, kB = chunked(src, 128, 0)
    dstB, _ = chunked(dst, 128, n)

    n_pad = _cdiv(n + 1, NS * 8) * NS * 8

    deg_parts = _sc_degree(dstB, n_pad, kB)
    dinv2d = _tc_dinv(deg_parts.reshape(NC, n_pad // 128, 128))
    dinv_col = dinv2d.reshape(n_pad)[:n].reshape(n, 1)

    y1 = _tc_matmul_scale(x, W1, dinv_col)
    agg1 = _sc_aggregate(srcA, dstA, y1, n, n_pad, kA, 64, 3)
    y2 = _tc_mid(agg1[:n], agg1[n_pad:n_pad + n], y1, dinv_col,
                 b1.reshape(1, -1), W2)
    agg2 = _sc_aggregate(srcB, dstB, y2, n, n_pad, kB, 128, 3)
    out = _tc_final(agg2[:n], agg2[n_pad:n_pad + n], y2, dinv_col,
                    b2.reshape(1, -1))
    return out


# speedup vs baseline: 22.8315x; 1.0428x over previous
import functools

import jax
import jax.numpy as jnp
from jax import lax
from jax.experimental import pallas as pl
from jax.experimental.pallas import tpu as pltpu
from jax.experimental.pallas import tpu_sc as plsc

NC = 2
NS = 16
NW = NC * NS
CH = 128
LANES = 16


def _cdiv(a, b):
    return (a + b - 1) // b


def _sc_degree(dst2d, n_pad, k_per_tile):
    mesh = plsc.VectorSubcoreMesh(core_axis_name="c", subcore_axis_name="s")
    zslice = n_pad // NS

    @functools.partial(
        pl.kernel,
        out_type=jax.ShapeDtypeStruct((NC * n_pad,), jnp.float32),
        mesh=mesh,
        scratch_types=[
            pltpu.VMEM((k_per_tile, CH), jnp.int32),
            pltpu.VMEM((CH,), jnp.float32),
            pltpu.VMEM((_cdiv(zslice, LANES) * LANES,), jnp.float32),
            pltpu.VMEM_SHARED((n_pad,), jnp.float32),
            pltpu.SemaphoreType.DMA,
        ],
        compiler_params=pltpu.CompilerParams(use_tc_tiling_on_sc=False),
    )
    def deg_kernel(dst_hbm, out_hbm, didx, ones, zbuf, acc, sem):
        c = lax.axis_index("c")
        s = lax.axis_index("s")
        wid = c * NS + s

        def fill_ones(i, carry):
            ones[pl.ds(i * LANES, LANES)] = jnp.full((LANES,), 1.0, jnp.float32)
            return carry

        lax.fori_loop(0, CH // LANES, fill_ones, 0)

        def fill_zero(i, carry):
            zbuf[pl.ds(i * LANES, LANES)] = jnp.zeros((LANES,), jnp.float32)
            return carry

        lax.fori_loop(0, _cdiv(zslice, LANES), fill_zero, 0)
        pltpu.sync_copy(zbuf.at[pl.ds(0, zslice)],
                        acc.at[pl.ds(s * zslice, zslice)])
        pltpu.sync_copy(dst_hbm.at[pl.ds(wid * k_per_tile, k_per_tile)], didx)
        plsc.subcore_barrier()

        BATCH = 8

        def batch(q, carry):
            for b in range(BATCH):
                k = q * BATCH + b

                @pl.when(k < k_per_tile)
                def _():
                    pltpu.async_copy(ones, acc.at[didx.at[k]], sem, add=True)
            for b in range(BATCH):
                k = q * BATCH + b

                @pl.when(k < k_per_tile)
                def _():
                    pltpu.make_async_copy(ones, acc.at[didx.at[k]],
                                          sem).wait()
            return carry

        lax.fori_loop(0, _cdiv(k_per_tile, BATCH), batch, 0)
        plsc.subcore_barrier()
        pltpu.sync_copy(acc.at[pl.ds(s * zslice, zslice)],
                        out_hbm.at[pl.ds(c * n_pad + s * zslice, zslice)])

    return deg_kernel(dst2d)


def _sc_aggregate(src2d, dst2d, y, n, n_pad, k_per_tile, ch, nbuf):
    d = y.shape[1]
    mesh = plsc.VectorSubcoreMesh(core_axis_name="c", subcore_axis_name="s")
    zslice = n_pad // NS

    @functools.partial(
        pl.kernel,
        out_type=jax.ShapeDtypeStruct((NC * n_pad, d), jnp.float32),
        mesh=mesh,
        scratch_types=[
            pltpu.VMEM((k_per_tile, ch), jnp.int32),
            pltpu.VMEM((k_per_tile, ch), jnp.int32),
            [pltpu.VMEM((ch, d), jnp.float32)] * nbuf,
            pltpu.VMEM_SHARED((n_pad, d), jnp.float32),
            [pltpu.SemaphoreType.DMA] * nbuf,
            [pltpu.SemaphoreType.DMA] * nbuf,
        ],
        compiler_params=pltpu.CompilerParams(use_tc_tiling_on_sc=False),
    )
    def agg_kernel(src_hbm, dst_hbm, y_hbm, out_hbm, sidx, didx, rows, acc,
                   gsem, ssem):
        c = lax.axis_index("c")
        s = lax.axis_index("s")
        wid = c * NS + s
        base = wid * k_per_tile

        def fill_zero(i, carry):
            j = i // (d // LANES)
            l = i % (d // LANES)
            rows[0][j, pl.ds(l * LANES, LANES)] = jnp.zeros((LANES,),
                                                            jnp.float32)
            return carry

        lax.fori_loop(0, ch * d // LANES, fill_zero, 0)

        def zero_acc(j, carry):
            pltpu.sync_copy(rows[0], acc.at[pl.ds(s * zslice + j * ch, ch)])
            return carry

        lax.fori_loop(0, zslice // ch, zero_acc, 0)
        rem = zslice % ch
        if rem:
            pltpu.sync_copy(
                rows[0].at[pl.ds(0, rem)],
                acc.at[pl.ds(s * zslice + (zslice // ch) * ch, rem)])
        pltpu.sync_copy(src_hbm.at[pl.ds(base, k_per_tile)], sidx)
        pltpu.sync_copy(dst_hbm.at[pl.ds(base, k_per_tile)], didx)
        plsc.subcore_barrier()

        def gather(k, b):
            pltpu.async_copy(y_hbm.at[sidx.at[k]], rows[b], gsem[b])

        def gather_wait(k, b):
            pltpu.make_async_copy(y_hbm.at[sidx.at[k]], rows[b],
                                  gsem[b]).wait()

        def scatter(k, b):
            pltpu.async_copy(rows[b], acc.at[didx.at[k]], ssem[b], add=True)

        def scatter_wait(k, b):
            pltpu.make_async_copy(rows[b], acc.at[didx.at[k]],
                                  ssem[b]).wait()

        for b in range(min(nbuf, k_per_tile)):
            gather(b, b)

        def ring(q, carry):
            for b in range(nbuf):
                k = q * nbuf + b

                @pl.when(k < k_per_tile)
                def _():
                    gather_wait(k, b)
                    scatter(k, b)
            for b in range(nbuf):
                k = q * nbuf + b

                @pl.when(k < k_per_tile)
                def _():
                    scatter_wait(k, b)

                    @pl.when(k + nbuf < k_per_tile)
                    def _():
                        gather(k + nbuf, b)
            return carry

        lax.fori_loop(0, _cdiv(k_per_tile, nbuf), ring, 0)
        plsc.subcore_barrier()
        pltpu.sync_copy(acc.at[pl.ds(s * zslice, zslice)],
                        out_hbm.at[pl.ds(c * n_pad + s * zslice, zslice)])

    return agg_kernel(src2d, dst2d, y)


def _tc_first(x, w, deg_a, deg_b):
    n = x.shape[0]

    def body(x_ref, w_ref, da_ref, db_ref, y_ref, v_ref):
        dinv = lax.rsqrt(da_ref[...] + db_ref[...] + 1.0)
        v_ref[...] = dinv
        y_ref[...] = jnp.dot(x_ref[...], w_ref[...],
                             preferred_element_type=jnp.float32) * dinv

    return pl.pallas_call(
        body,
        out_shape=[jax.ShapeDtypeStruct((n, w.shape[1]), jnp.float32),
                   jax.ShapeDtypeStruct((n, 1), jnp.float32)],
    )(x, w, deg_a, deg_b)


def _tc_mid(agg, y1, dinv_col, b1row, w2, n, n_pad):
    def body(agg_ref, y_ref, v_ref, bias_ref, w_ref, o_ref):
        acc = agg_ref[pl.ds(0, n), :] + agg_ref[pl.ds(n_pad, n), :]
        h = (acc + y_ref[...]) * v_ref[...] + bias_ref[...]
        h = jnp.maximum(h, 0.0)
        o_ref[...] = jnp.dot(h, w_ref[...],
                             preferred_element_type=jnp.float32) * v_ref[...]

    return pl.pallas_call(
        body,
        out_shape=jax.ShapeDtypeStruct((n, w2.shape[1]), jnp.float32),
    )(agg, y1, dinv_col, b1row, w2)


def _tc_final(agg, y2, dinv_col, b2row, n, n_pad):
    def body(agg_ref, y_ref, v_ref, bias_ref, o_ref):
        acc = agg_ref[pl.ds(0, n), :] + agg_ref[pl.ds(n_pad, n), :]
        o_ref[...] = (acc + y_ref[...]) * v_ref[...] + bias_ref[...]

    return pl.pallas_call(
        body,
        out_shape=jax.ShapeDtypeStruct(y2.shape, jnp.float32),
    )(agg, y2, dinv_col, b2row)


def kernel(x, edge_index, W1, b1, W2, b2):
    n = x.shape[0]
    e = edge_index.shape[1]

    src = edge_index[0].astype(jnp.int32)
    dst = edge_index[1].astype(jnp.int32)

    def chunked(arr, ch, fill):
        k_per_tile = _cdiv(_cdiv(e, ch), NW)
        padn = k_per_tile * NW * ch - e
        return (jnp.concatenate(
            [arr, jnp.full((padn,), fill, jnp.int32)]).reshape(
                k_per_tile * NW, ch), k_per_tile)

    srcA, kA = chunked(src, 64, 0)
    dstA, _ = chunked(dst, 64, n)
    srcB, kB = chunked(src, 128, 0)
    dstB, _ = chunked(dst, 128, n)

    n_pad = _cdiv(n + 1, NS * 8) * NS * 8

    deg_parts = _sc_degree(dstB, n_pad, kB)
    deg_a = deg_parts[:n].reshape(n, 1)
    deg_b = deg_parts[n_pad:n_pad + n].reshape(n, 1)

    y1, dinv_col = _tc_first(x, W1, deg_a, deg_b)
    agg1 = _sc_aggregate(srcA, dstA, y1, n, n_pad, kA, 64, 3)
    y2 = _tc_mid(agg1, y1, dinv_col, b1.reshape(1, -1), W2, n, n_pad)
    agg2 = _sc_aggregate(srcB, dstB, y2, n, n_pad, kB, 128, 6)
    out = _tc_final(agg2, y2, dinv_col, b2.reshape(1, -1), n, n_pad)
    return out


# speedup vs baseline: 29.9628x; 1.3123x over previous
import functools

import jax
import jax.numpy as jnp
from jax import lax
from jax.experimental import pallas as pl
from jax.experimental.pallas import tpu as pltpu
from jax.experimental.pallas import tpu_sc as plsc

NC = 2
NS = 16
NW = NC * NS
CH = 128
LANES = 16


def _cdiv(a, b):
    return (a + b - 1) // b


def _sc_degree(dst2d, n_pad, k0, k1):
    mesh = plsc.VectorSubcoreMesh(core_axis_name="c", subcore_axis_name="s")
    zslice = n_pad // NS
    kmax = max(k0, k1)

    @functools.partial(
        pl.kernel,
        out_type=jax.ShapeDtypeStruct((NC * n_pad,), jnp.float32),
        mesh=mesh,
        scratch_types=[
            pltpu.VMEM((kmax, CH), jnp.int32),
            pltpu.VMEM((CH,), jnp.float32),
            pltpu.VMEM((_cdiv(zslice, LANES) * LANES,), jnp.float32),
            pltpu.VMEM_SHARED((n_pad,), jnp.float32),
            pltpu.SemaphoreType.DMA,
        ],
        compiler_params=pltpu.CompilerParams(use_tc_tiling_on_sc=False),
    )
    def deg_kernel(dst_hbm, out_hbm, didx, ones, zbuf, acc, sem):
        c = lax.axis_index("c")
        s = lax.axis_index("s")
        kpt = jnp.where(c == 0, k0, k1)
        base = jnp.where(c == 0, s * k0, NS * k0 + s * k1)

        def fill_ones(i, carry):
            ones[pl.ds(i * LANES, LANES)] = jnp.full((LANES,), 1.0, jnp.float32)
            return carry

        lax.fori_loop(0, CH // LANES, fill_ones, 0)

        def fill_zero(i, carry):
            zbuf[pl.ds(i * LANES, LANES)] = jnp.zeros((LANES,), jnp.float32)
            return carry

        lax.fori_loop(0, _cdiv(zslice, LANES), fill_zero, 0)
        pltpu.sync_copy(zbuf.at[pl.ds(0, zslice)],
                        acc.at[pl.ds(s * zslice, zslice)])
        pltpu.sync_copy(dst_hbm.at[pl.ds(base, kmax)], didx)
        plsc.subcore_barrier()

        BATCH = 8

        def batch(q, carry):
            for b in range(BATCH):
                k = q * BATCH + b

                @pl.when(k < kpt)
                def _():
                    pltpu.async_copy(ones, acc.at[didx.at[k]], sem, add=True)
            for b in range(BATCH):
                k = q * BATCH + b

                @pl.when(k < kpt)
                def _():
                    pltpu.make_async_copy(ones, acc.at[didx.at[k]],
                                          sem).wait()
            return carry

        lax.fori_loop(0, (kpt + BATCH - 1) // BATCH, batch, 0)
        plsc.subcore_barrier()
        pltpu.sync_copy(acc.at[pl.ds(s * zslice, zslice)],
                        out_hbm.at[pl.ds(c * n_pad + s * zslice, zslice)])

    return deg_kernel(dst2d)


def _sc_aggregate(src2d, dst2d, y, n, n_pad, k0, k1, ch, nbuf):
    d = y.shape[1]
    mesh = plsc.VectorSubcoreMesh(core_axis_name="c", subcore_axis_name="s")
    zslice = n_pad // NS
    kmax = max(k0, k1)

    @functools.partial(
        pl.kernel,
        out_type=jax.ShapeDtypeStruct((NC * n_pad, d), jnp.float32),
        mesh=mesh,
        scratch_types=[
            pltpu.VMEM((kmax, ch), jnp.int32),
            pltpu.VMEM((kmax, ch), jnp.int32),
            [pltpu.VMEM((ch, d), jnp.float32)] * nbuf,
            pltpu.VMEM_SHARED((n_pad, d), jnp.float32),
            [pltpu.SemaphoreType.DMA] * nbuf,
            [pltpu.SemaphoreType.DMA] * nbuf,
        ],
        compiler_params=pltpu.CompilerParams(use_tc_tiling_on_sc=False),
    )
    def agg_kernel(src_hbm, dst_hbm, y_hbm, out_hbm, sidx, didx, rows, acc,
                   gsem, ssem):
        c = lax.axis_index("c")
        s = lax.axis_index("s")
        kpt = jnp.where(c == 0, k0, k1)
        base = jnp.where(c == 0, s * k0, NS * k0 + s * k1)

        def fill_zero(i, carry):
            j = i // (d // LANES)
            l = i % (d // LANES)
            rows[0][j, pl.ds(l * LANES, LANES)] = jnp.zeros((LANES,),
                                                            jnp.float32)
            return carry

        lax.fori_loop(0, ch * d // LANES, fill_zero, 0)

        def zero_acc(j, carry):
            pltpu.sync_copy(rows[0], acc.at[pl.ds(s * zslice + j * ch, ch)])
            return carry

        lax.fori_loop(0, zslice // ch, zero_acc, 0)
        rem = zslice % ch
        if rem:
            pltpu.sync_copy(
                rows[0].at[pl.ds(0, rem)],
                acc.at[pl.ds(s * zslice + (zslice // ch) * ch, rem)])
        pltpu.sync_copy(src_hbm.at[pl.ds(base, kmax)], sidx)
        pltpu.sync_copy(dst_hbm.at[pl.ds(base, kmax)], didx)
        plsc.subcore_barrier()

        def gather(k, b):
            pltpu.async_copy(y_hbm.at[sidx.at[k]], rows[b], gsem[b])

        def gather_wait(k, b):
            pltpu.make_async_copy(y_hbm.at[sidx.at[k]], rows[b],
                                  gsem[b]).wait()

        def scatter(k, b):
            pltpu.async_copy(rows[b], acc.at[didx.at[k]], ssem[b], add=True)

        def scatter_wait(k, b):
            pltpu.make_async_copy(rows[b], acc.at[didx.at[k]],
                                  ssem[b]).wait()

        for b in range(nbuf):
            @pl.when(b < kpt)
            def _():
                gather(b, b)

        def ring(q, carry):
            for b in range(nbuf):
                k = q * nbuf + b

                @pl.when(k < kpt)
                def _():
                    gather_wait(k, b)
                    scatter(k, b)
            for b in range(nbuf):
                k = q * nbuf + b

                @pl.when(k < kpt)
                def _():
                    scatter_wait(k, b)

                    @pl.when(k + nbuf < kpt)
                    def _():
                        gather(k + nbuf, b)
            return carry

        lax.fori_loop(0, (kpt + nbuf - 1) // nbuf, ring, 0)
        plsc.subcore_barrier()
        pltpu.sync_copy(acc.at[pl.ds(s * zslice, zslice)],
                        out_hbm.at[pl.ds(c * n_pad + s * zslice, zslice)])

    return agg_kernel(src2d, dst2d, y)


def _tc_first(x, w, deg_a, deg_b):
    n = x.shape[0]

    def body(x_ref, w_ref, da_ref, db_ref, y_ref, v_ref):
        dinv = lax.rsqrt(da_ref[...] + db_ref[...] + 1.0)
        v_ref[...] = dinv
        y_ref[...] = jnp.dot(x_ref[...], w_ref[...],
                             preferred_element_type=jnp.float32) * dinv

    return pl.pallas_call(
        body,
        out_shape=[jax.ShapeDtypeStruct((n, w.shape[1]), jnp.float32),
                   jax.ShapeDtypeStruct((n, 1), jnp.float32)],
    )(x, w, deg_a, deg_b)


def _tc_mid(agg, y1, dinv_col, b1row, w2, n, n_pad):
    def body(agg_ref, y_ref, v_ref, bias_ref, w_ref, o_ref):
        acc = agg_ref[pl.ds(0, n), :] + agg_ref[pl.ds(n_pad, n), :]
        h = (acc + y_ref[...]) * v_ref[...] + bias_ref[...]
        h = jnp.maximum(h, 0.0)
        o_ref[...] = jnp.dot(h, w_ref[...],
                             preferred_element_type=jnp.float32) * v_ref[...]

    return pl.pallas_call(
        body,
        out_shape=jax.ShapeDtypeStruct((n, w2.shape[1]), jnp.float32),
    )(agg, y1, dinv_col, b1row, w2)


def _tc_final(agg, y2, dinv_col, b2row, n, n_pad):
    def body(agg_ref, y_ref, v_ref, bias_ref, o_ref):
        acc = agg_ref[pl.ds(0, n), :] + agg_ref[pl.ds(n_pad, n), :]
        o_ref[...] = (acc + y_ref[...]) * v_ref[...] + bias_ref[...]

    return pl.pallas_call(
        body,
        out_shape=jax.ShapeDtypeStruct(y2.shape, jnp.float32),
    )(agg, y2, dinv_col, b2row)


def kernel(x, edge_index, W1, b1, W2, b2):
    n = x.shape[0]
    e = edge_index.shape[1]

    src = edge_index[0].astype(jnp.int32)
    dst = edge_index[1].astype(jnp.int32)

    def chunked(arr, ch, fill, f0):
        n_chunks = _cdiv(e, ch)
        c0 = int(n_chunks * f0)
        k0 = max(_cdiv(c0, NS), 1)
        k1 = max(_cdiv(n_chunks - NS * k0, NS), 1)
        rows = NS * (k0 + k1) + abs(k0 - k1)
        padn = rows * ch - e
        return (jnp.concatenate(
            [arr, jnp.full((padn,), fill, jnp.int32)]).reshape(rows, ch),
            k0, k1)

    F0_A = 0.6
    F0_B = 0.6
    srcA, kA0, kA1 = chunked(src, 64, 0, F0_A)
    dstA, _, _ = chunked(dst, 64, n, F0_A)
    srcB, kB0, kB1 = chunked(src, 128, 0, F0_B)
    dstB, _, _ = chunked(dst, 128, n, F0_B)

    n_pad = _cdiv(n + 1, NS * 8) * NS * 8

    deg_parts = _sc_degree(dstB, n_pad, kB0, kB1)
    deg_a = deg_parts[:n].reshape(n, 1)
    deg_b = deg_parts[n_pad:n_pad + n].reshape(n, 1)

    y1, dinv_col = _tc_first(x, W1, deg_a, deg_b)
    agg1 = _sc_aggregate(srcA, dstA, y1, n, n_pad, kA0, kA1, 64, 3)
    y2 = _tc_mid(agg1, y1, dinv_col, b1.reshape(1, -1), W2, n, n_pad)
    agg2 = _sc_aggregate(srcB, dstB, y2, n, n_pad, kB0, kB1, 128, 6)
    out = _tc_final(agg2, y2, dinv_col, b2.reshape(1, -1), n, n_pad)
    return out


# speedup vs baseline: 30.2746x; 1.0104x over previous
import functools

import jax
import jax.numpy as jnp
from jax import lax
from jax.experimental import pallas as pl
from jax.experimental.pallas import tpu as pltpu
from jax.experimental.pallas import tpu_sc as plsc

NC = 2
NS = 16
NW = NC * NS
CH = 128
LANES = 16


def _cdiv(a, b):
    return (a + b - 1) // b


def _sc_degree(dst2d, n_pad, k0, k1):
    mesh = plsc.VectorSubcoreMesh(core_axis_name="c", subcore_axis_name="s")
    zslice = n_pad // NS
    kmax = max(k0, k1)

    @functools.partial(
        pl.kernel,
        out_type=jax.ShapeDtypeStruct((NC * n_pad,), jnp.float32),
        mesh=mesh,
        scratch_types=[
            pltpu.VMEM((kmax, CH), jnp.int32),
            pltpu.VMEM((CH,), jnp.float32),
            pltpu.VMEM((_cdiv(zslice, LANES) * LANES,), jnp.float32),
            pltpu.VMEM_SHARED((n_pad,), jnp.float32),
            pltpu.SemaphoreType.DMA,
        ],
        compiler_params=pltpu.CompilerParams(use_tc_tiling_on_sc=False),
    )
    def deg_kernel(dst_hbm, out_hbm, didx, ones, zbuf, acc, sem):
        c = lax.axis_index("c")
        s = lax.axis_index("s")
        kpt = jnp.where(c == 0, k0, k1)
        base = jnp.where(c == 0, s * k0, NS * k0 + s * k1)

        def fill_ones(i, carry):
            ones[pl.ds(i * LANES, LANES)] = jnp.full((LANES,), 1.0, jnp.float32)
            return carry

        lax.fori_loop(0, CH // LANES, fill_ones, 0)

        def fill_zero(i, carry):
            zbuf[pl.ds(i * LANES, LANES)] = jnp.zeros((LANES,), jnp.float32)
            return carry

        lax.fori_loop(0, _cdiv(zslice, LANES), fill_zero, 0)
        pltpu.sync_copy(zbuf.at[pl.ds(0, zslice)],
                        acc.at[pl.ds(s * zslice, zslice)])
        pltpu.sync_copy(dst_hbm.at[pl.ds(base, kmax)], didx)
        plsc.subcore_barrier()

        BATCH = 8

        def batch(q, carry):
            for b in range(BATCH):
                k = q * BATCH + b

                @pl.when(k < kpt)
                def _():
                    pltpu.async_copy(ones, acc.at[didx.at[k]], sem, add=True)
            for b in range(BATCH):
                k = q * BATCH + b

                @pl.when(k < kpt)
                def _():
                    pltpu.make_async_copy(ones, acc.at[didx.at[k]],
                                          sem).wait()
            return carry

        lax.fori_loop(0, (kpt + BATCH - 1) // BATCH, batch, 0)
        plsc.subcore_barrier()
        pltpu.sync_copy(acc.at[pl.ds(s * zslice, zslice)],
                        out_hbm.at[pl.ds(c * n_pad + s * zslice, zslice)])

    return deg_kernel(dst2d)


def _sc_aggregate(src2d, dst2d, y, n, n_pad, k0, k1, ch, nbuf):
    d = y.shape[1]
    mesh = plsc.VectorSubcoreMesh(core_axis_name="c", subcore_axis_name="s")
    zslice = n_pad // NS
    kmax = max(k0, k1)

    @functools.partial(
        pl.kernel,
        out_type=jax.ShapeDtypeStruct((NC * n_pad, d), jnp.float32),
        mesh=mesh,
        scratch_types=[
            pltpu.VMEM((kmax, ch), jnp.int32),
            pltpu.VMEM((kmax, ch), jnp.int32),
            [pltpu.VMEM((ch, d), jnp.float32)] * nbuf,
            pltpu.VMEM_SHARED((n_pad, d), jnp.float32),
            [pltpu.SemaphoreType.DMA] * nbuf,
            [pltpu.SemaphoreType.DMA] * nbuf,
        ],
        compiler_params=pltpu.CompilerParams(use_tc_tiling_on_sc=False),
    )
    def agg_kernel(src_hbm, dst_hbm, y_hbm, out_hbm, sidx, didx, rows, acc,
                   gsem, ssem):
        c = lax.axis_index("c")
        s = lax.axis_index("s")
        kpt = jnp.where(c == 0, k0, k1)
        base = jnp.where(c == 0, s * k0, NS * k0 + s * k1)

        def fill_zero(i, carry):
            j = i // (d // LANES)
            l = i % (d // LANES)
            rows[0][j, pl.ds(l * LANES, LANES)] = jnp.zeros((LANES,),
                                                            jnp.float32)
            return carry

        lax.fori_loop(0, ch * d // LANES, fill_zero, 0)

        def zero_acc(j, carry):
            pltpu.sync_copy(rows[0], acc.at[pl.ds(s * zslice + j * ch, ch)])
            return carry

        lax.fori_loop(0, zslice // ch, zero_acc, 0)
        rem = zslice % ch
        if rem:
            pltpu.sync_copy(
                rows[0].at[pl.ds(0, rem)],
                acc.at[pl.ds(s * zslice + (zslice // ch) * ch, rem)])
        pltpu.sync_copy(src_hbm.at[pl.ds(base, kmax)], sidx)
        pltpu.sync_copy(dst_hbm.at[pl.ds(base, kmax)], didx)
        plsc.subcore_barrier()

        def gather(k, b):
            pltpu.async_copy(y_hbm.at[sidx.at[k]], rows[b], gsem[b])

        def gather_wait(k, b):
            pltpu.make_async_copy(y_hbm.at[sidx.at[k]], rows[b],
                                  gsem[b]).wait()

        def scatter(k, b):
            pltpu.async_copy(rows[b], acc.at[didx.at[k]], ssem[b], add=True)

        def scatter_wait(k, b):
            pltpu.make_async_copy(rows[b], acc.at[didx.at[k]],
                                  ssem[b]).wait()

        for b in range(nbuf):
            @pl.when(b < kpt)
            def _():
                gather(b, b)

        def ring(q, carry):
            for b in range(nbuf):
                k = q * nbuf + b

                @pl.when(k < kpt)
                def _():
                    gather_wait(k, b)
                    scatter(k, b)
            for b in range(nbuf):
                k = q * nbuf + b

                @pl.when(k < kpt)
                def _():
                    scatter_wait(k, b)

                    @pl.when(k + nbuf < kpt)
                    def _():
                        gather(k + nbuf, b)
            return carry

        lax.fori_loop(0, (kpt + nbuf - 1) // nbuf, ring, 0)
        plsc.subcore_barrier()
        pltpu.sync_copy(acc.at[pl.ds(s * zslice, zslice)],
                        out_hbm.at[pl.ds(c * n_pad + s * zslice, zslice)])

    return agg_kernel(src2d, dst2d, y)


def _tc_first(x, w, deg_a, deg_b):
    n = x.shape[0]

    def body(x_ref, w_ref, da_ref, db_ref, y_ref, v_ref):
        dinv = lax.rsqrt(da_ref[...] + db_ref[...] + 1.0)
        v_ref[...] = dinv
        y_ref[...] = jnp.dot(x_ref[...], w_ref[...],
                             preferred_element_type=jnp.float32) * dinv

    return pl.pallas_call(
        body,
        out_shape=[jax.ShapeDtypeStruct((n, w.shape[1]), jnp.float32),
                   jax.ShapeDtypeStruct((n, 1), jnp.float32)],
    )(x, w, deg_a, deg_b)


def _tc_mid(agg, y1, dinv_col, b1row, w2, n, n_pad):
    def body(agg_ref, y_ref, v_ref, bias_ref, w_ref, o_ref):
        acc = agg_ref[pl.ds(0, n), :] + agg_ref[pl.ds(n_pad, n), :]
        h = (acc + y_ref[...]) * v_ref[...] + bias_ref[...]
        h = jnp.maximum(h, 0.0)
        o_ref[...] = jnp.dot(h, w_ref[...],
                             preferred_element_type=jnp.float32) * v_ref[...]

    return pl.pallas_call(
        body,
        out_shape=jax.ShapeDtypeStruct((n, w2.shape[1]), jnp.float32),
    )(agg, y1, dinv_col, b1row, w2)


def _tc_final(agg, y2, dinv_col, b2row, n, n_pad):
    def body(agg_ref, y_ref, v_ref, bias_ref, o_ref):
        acc = agg_ref[pl.ds(0, n), :] + agg_ref[pl.ds(n_pad, n), :]
        o_ref[...] = (acc + y_ref[...]) * v_ref[...] + bias_ref[...]

    return pl.pallas_call(
        body,
        out_shape=jax.ShapeDtypeStruct(y2.shape, jnp.float32),
    )(agg, y2, dinv_col, b2row)


def kernel(x, edge_index, W1, b1, W2, b2):
    n = x.shape[0]
    e = edge_index.shape[1]

    src = edge_index[0].astype(jnp.int32)
    dst = edge_index[1].astype(jnp.int32)

    def chunked(arr, ch, fill, f0):
        n_chunks = _cdiv(e, ch)
        c0 = int(n_chunks * f0)
        k0 = max(_cdiv(c0, NS), 1)
        k1 = max(_cdiv(n_chunks - NS * k0, NS), 1)
        rows = NS * (k0 + k1) + abs(k0 - k1)
        padn = rows * ch - e
        return (jnp.concatenate(
            [arr, jnp.full((padn,), fill, jnp.int32)]).reshape(rows, ch),
            k0, k1)

    F0_A = 0.6
    F0_B = 0.65
    srcA, kA0, kA1 = chunked(src, 64, 0, F0_A)
    dstA, _, _ = chunked(dst, 64, n, F0_A)
    srcB, kB0, kB1 = chunked(src, 128, 0, F0_B)
    dstB, _, _ = chunked(dst, 128, n, F0_B)

    n_pad = _cdiv(n + 1, NS * 8) * NS * 8

    deg_parts = _sc_degree(dstB, n_pad, kB0, kB1)
    deg_a = deg_parts[:n].reshape(n, 1)
    deg_b = deg_parts[n_pad:n_pad + n].reshape(n, 1)

    y1, dinv_col = _tc_first(x, W1, deg_a, deg_b)
    agg1 = _sc_aggregate(srcA, dstA, y1, n, n_pad, kA0, kA1, 64, 3)
    y2 = _tc_mid(agg1, y1, dinv_col, b1.reshape(1, -1), W2, n, n_pad)
    agg2 = _sc_aggregate(srcB, dstB, y2, n, n_pad, kB0, kB1, 128, 6)
    out = _tc_final(agg2, y2, dinv_col, b2.reshape(1, -1), n, n_pad)
    return out
